# Initial kernel scaffold; baseline (speedup 1.0000x reference)
import functools

import jax
import jax.numpy as jnp
from jax import lax
from jax.experimental import pallas as pl
from jax.experimental.pallas import tpu as pltpu
from jax.experimental.pallas import tpu_sc as plsc

N = 10000
E = 320000
D = 128
G = 16
NC = 2
NS = 16
NW = NC * NS
CW = 128
NCHUNK = 80
EPW = NCHUNK * CW
EPAD = NW * EPW
NPAD = 10240
RPT = NPAD // NS
EPW_U = E // NW
NEG = -jnp.inf


@functools.partial(
    pl.kernel,
    out_type=jax.ShapeDtypeStruct((NC, NPAD, D), jnp.float32),
    mesh=plsc.VectorSubcoreMesh(core_axis_name="c", subcore_axis_name="s",
                                num_cores=NC, num_subcores=NS),
    scratch_types=[
        pltpu.VMEM((NCHUNK, CW), jnp.int32),
        pltpu.VMEM((NCHUNK, CW), jnp.int32),
        pltpu.VMEM((CW, D), jnp.float32),
        pltpu.VMEM_SHARED((NPAD, D), jnp.float32),
        pltpu.SemaphoreType.DMA,
    ],
)
def _sc_scatter_rows(h_hbm, src_hbm, dst_hbm, z_hbm, out_hbm,
                     src_v, dst_v, rows_v, acc_sh, sem):
    c = lax.axis_index("c")
    s = lax.axis_index("s")
    w = s * NC + c
    pltpu.sync_copy(z_hbm, acc_sh.at[pl.ds(s * RPT, RPT)])
    pltpu.sync_copy(src_hbm.at[w], src_v)
    pltpu.sync_copy(dst_hbm.at[w], dst_v)
    plsc.subcore_barrier()

    def body(j, carry):
        pltpu.async_copy(h_hbm.at[src_v.at[j]], rows_v, sem).wait()
        pltpu.sync_copy(rows_v, acc_sh.at[dst_v.at[j]], add=True)
        return carry

    lax.fori_loop(0, NCHUNK, body, 0)
    plsc.subcore_barrier()
    pltpu.sync_copy(acc_sh.at[pl.ds(s * RPT, RPT)],
                    out_hbm.at[c, pl.ds(s * RPT, RPT)])


@functools.partial(
    pl.kernel,
    out_type=jax.ShapeDtypeStruct((10, NW, 1000), jnp.float32),
    mesh=plsc.VectorSubcoreMesh(core_axis_name="c", subcore_axis_name="s",
                                num_cores=NC, num_subcores=NS),
    scratch_types=[
        pltpu.VMEM((EPW_U,), jnp.int32),
        pltpu.VMEM((N,), jnp.float32),
    ],
)
def _sc_counts(dst_hbm, out_hbm, dst_v, acc_v):
    c = lax.axis_index("c")
    s = lax.axis_index("s")
    w = s * NC + c
    pltpu.sync_copy(dst_hbm.at[w], dst_v)
    zeros16 = jnp.zeros((16,), jnp.float32)

    def zbody(i, carry):
        acc_v[pl.ds(i * 16, 16)] = zeros16
        return carry

    lax.fori_loop(0, N // 16, zbody, 0)
    ones16 = jnp.ones((16,), jnp.float32)

    def body(j, carry):
        idx = dst_v[pl.ds(j * 16, 16)]
        plsc.addupdate_scatter(acc_v, [idx], ones16)
        return carry

    lax.fori_loop(0, EPW_U // 16, body, 0)
    for r in range(10):
        pltpu.sync_copy(acc_v.at[pl.ds(r * 1000, 1000)], out_hbm.at[r, w])


@functools.partial(
    pl.kernel,
    out_type=jax.ShapeDtypeStruct((10, NW, 1000), jnp.float32),
    mesh=plsc.VectorSubcoreMesh(core_axis_name="c", subcore_axis_name="s",
                                num_cores=NC, num_subcores=NS),
    scratch_types=[
        pltpu.VMEM((N,), jnp.float32),
        pltpu.VMEM((EPW_U,), jnp.int32),
        pltpu.VMEM((EPW_U,), jnp.int32),
        pltpu.VMEM((N,), jnp.float32),
    ],
)
def _sc_edge_scalar(q_hbm, src_hbm, dst_hbm, out_hbm, q_v, src_v, dst_v, acc_v):
    c = lax.axis_index("c")
    s = lax.axis_index("s")
    w = s * NC + c
    pltpu.sync_copy(q_hbm, q_v)
    pltpu.sync_copy(src_hbm.at[w], src_v)
    pltpu.sync_copy(dst_hbm.at[w], dst_v)
    zeros16 = jnp.zeros((16,), jnp.float32)

    def zbody(i, carry):
        acc_v[pl.ds(i * 16, 16)] = zeros16
        return carry

    lax.fori_loop(0, N // 16, zbody, 0)

    def body(j, carry):
        si = src_v[pl.ds(j * 16, 16)]
        di = dst_v[pl.ds(j * 16, 16)]
        vals = plsc.load_gather(q_v, [si])
        plsc.addupdate_scatter(acc_v, [di], vals)
        return carry

    lax.fori_loop(0, EPW_U // 16, body, 0)
    for r in range(10):
        pltpu.sync_copy(acc_v.at[pl.ds(r * 1000, 1000)], out_hbm.at[r, w])


def _gmax_update(gm_ref, hn, bb, r):
    prev = jnp.where(r == 0, jnp.full((G, D), NEG, jnp.float32), gm_ref[...])
    cand = jnp.stack(
        [jnp.max(jnp.where((bb == g)[:, None], hn, NEG), axis=0)
         for g in range(G)], axis=0)
    gm_ref[...] = jnp.maximum(prev, cand)


def _make_conv_call(has_cnt, has_q):
    def body(*refs):
        it = iter(refs)
        p_ref = next(it); h_ref = next(it); b_ref = next(it)
        wl_ref = next(it); bl_ref = next(it); wr_ref = next(it)
        cnt_ref = next(it) if has_cnt else None
        c_ref = None if has_cnt else next(it)
        gcn_ref = next(it) if has_q else None
        hn_ref = next(it); gm_ref = next(it)
        c_out = next(it) if has_cnt else None
        q_out = next(it) if has_q else None

        r = pl.program_id(0)
        ssum = p_ref[0] + p_ref[1]
        if has_cnt:
            cvec = jnp.sum(cnt_ref[0], axis=0)
            c_out[0, 0] = cvec
        else:
            cvec = c_ref[0, 0]
        mean = ssum / jnp.maximum(cvec, 1.0)[:, None]
        hn = (jnp.dot(mean, wl_ref[...], preferred_element_type=jnp.float32)
              + bl_ref[...]
              + jnp.dot(h_ref[...], wr_ref[...],
                        preferred_element_type=jnp.float32))
        hn_ref[...] = hn
        _gmax_update(gm_ref, hn, b_ref[0, 0], r)
        if has_q:
            hl = jnp.dot(hn, gcn_ref[...], preferred_element_type=jnp.float32)
            dinv = lax.rsqrt(cvec + 1.0)
            q_out[0, 0] = hl[:, 0] * dinv

    in_specs = [
        pl.BlockSpec((2, 1000, D), lambda r: (0, r, 0)),
        pl.BlockSpec((1000, D), lambda r: (r, 0)),
        pl.BlockSpec((1, 1, 1000), lambda r: (r, 0, 0)),
        pl.BlockSpec((D, D), lambda r: (0, 0)),
        pl.BlockSpec((1, D), lambda r: (0, 0)),
        pl.BlockSpec((D, D), lambda r: (0, 0)),
    ]
    if has_cnt:
        in_specs.append(pl.BlockSpec((1, NW, 1000), lambda r: (r, 0, 0)))
    else:
        in_specs.append(pl.BlockSpec((1, 1, 1000), lambda r: (r, 0, 0)))
    if has_q:
        in_specs.append(pl.BlockSpec((D, 1), lambda r: (0, 0)))

    out_shapes = [jax.ShapeDtypeStruct((N, D), jnp.float32),
                  jax.ShapeDtypeStruct((G, D), jnp.float32)]
    out_specs = [pl.BlockSpec((1000, D), lambda r: (r, 0)),
                 pl.BlockSpec((G, D), lambda r: (0, 0))]
    if has_cnt:
        out_shapes.append(jax.ShapeDtypeStruct((10, 1, 1000), jnp.float32))
        out_specs.append(pl.BlockSpec((1, 1, 1000), lambda r: (r, 0, 0)))
    if has_q:
        out_shapes.append(jax.ShapeDtypeStruct((10, 1, 1000), jnp.float32))
        out_specs.append(pl.BlockSpec((1, 1, 1000), lambda r: (r, 0, 0)))

    return pl.pallas_call(
        body,
        grid=(10,),
        in_specs=in_specs,
        out_specs=tuple(out_specs),
        out_shape=tuple(out_shapes),
    )


_conv1_call = _make_conv_call(True, False)
_conv2_call = _make_conv_call(False, False)
_conv3_call = _make_conv_call(False, True)


def _scorepool_body(h3_ref, sp_ref, c_ref, q_ref, b_ref, gb_ref, gm_ref):
    es = jnp.sum(sp_ref[...], axis=1)
    cc = c_ref[...][:, 0, :]
    qq = q_ref[...][:, 0, :]
    bb = b_ref[...][:, 0, :]
    dinv = lax.rsqrt(cc + 1.0)
    sraw = dinv * (es + qq) + gb_ref[0, 0]
    masks = [bb == g for g in range(G)]
    smax_n = jnp.zeros_like(sraw)
    for g in range(G):
        mg = jnp.max(jnp.where(masks[g], sraw, NEG))
        smax_n = smax_n + jnp.where(masks[g], mg, 0.0)
    e = jnp.exp(sraw - smax_n)
    ssum_n = jnp.zeros_like(sraw)
    for g in range(G):
        sg = jnp.sum(jnp.where(masks[g], e, 0.0))
        ssum_n = ssum_n + jnp.where(masks[g], sg, 0.0)
    score = e / ssum_n
    scmax_n = jnp.zeros_like(sraw)
    for g in range(G):
        mg = jnp.max(jnp.where(masks[g], score, NEG))
        scmax_n = scmax_n + jnp.where(masks[g], mg, 0.0)
    scmin = jnp.minimum(scmax_n - 1e-7, 0.001)
    keep = score > scmin

    scf = score.reshape(N)
    keepf = keep.reshape(N)
    masked = jnp.where(keepf[:, None], h3_ref[...] * scf[:, None], NEG)
    bflat = bb.reshape(N)
    gm_ref[...] = jnp.stack(
        [jnp.max(jnp.where((bflat == g)[:, None], masked, NEG), axis=0)
         for g in range(G)], axis=0)


_scorepool_call = pl.pallas_call(
    _scorepool_body,
    out_shape=jax.ShapeDtypeStruct((G, D), jnp.float32),
)


def _head_body(g1_ref, g2_ref, g3_ref, g4_ref, w1_ref, b1_ref, bg_ref, bb_ref,
               w2_ref, b2_ref, y_ref, logits_ref, loss_ref):
    z = jnp.concatenate(
        [g1_ref[...], g2_ref[...], g3_ref[...], g4_ref[...]], axis=1)
    z = jnp.dot(z, w1_ref[...], preferred_element_type=jnp.float32) + b1_ref[...]
    mu = jnp.mean(z, axis=0, keepdims=True)
    var = jnp.mean((z - mu) ** 2, axis=0, keepdims=True)
    z = (z - mu) / jnp.sqrt(var + 1e-5) * bg_ref[...] + bb_ref[...]
    z = jnp.maximum(z, 0.0)
    logits = jnp.dot(z, w2_ref[...], preferred_element_type=jnp.float32) + b2_ref[...]
    m = jnp.max(logits, axis=1, keepdims=True)
    lse = m + jnp.log(jnp.sum(jnp.exp(logits - m), axis=1, keepdims=True))
    logp = logits - lse
    oh = (y_ref[...] == lax.broadcasted_iota(jnp.int32, (1, 2), 1)
          ).astype(jnp.float32)
    loss = -jnp.mean(jnp.sum(logp * oh, axis=1))
    logits_ref[...] = logits
    loss_ref[0, 0] = loss


_head_call = pl.pallas_call(
    _head_body,
    out_shape=(jax.ShapeDtypeStruct((G, 2), jnp.float32),
               jax.ShapeDtypeStruct((1, 1), jnp.float32)),
)


def kernel(x, edge_index, batch, y, conv_Wl, conv_bl, conv_Wr,
           conv1_Wl, conv1_bl, conv1_Wr, conv3_Wl, conv3_bl, conv3_Wr,
           gcn_W, gcn_b, fc1_W, fc1_b, bn_g, bn_b, fc2_W, fc2_b):
    src = edge_index[0]
    dst = edge_index[1]
    npad = EPAD - E
    pad_src = jnp.zeros((npad,), jnp.int32)
    pad_dst = N + (jnp.arange(npad, dtype=jnp.int32) % (NPAD - N))
    srcp = jnp.concatenate([src, pad_src]).reshape(NW, NCHUNK, CW)
    dstp = jnp.concatenate([dst, pad_dst]).reshape(NW, NCHUNK, CW)
    src_flat = src.reshape(NW, EPW_U)
    dst_flat = dst.reshape(NW, EPW_U)
    batch3 = batch.reshape(10, 1, 1000)
    zrows = jnp.zeros((RPT, D), jnp.float32)
    bl2 = conv_bl.reshape(1, D)
    b12 = conv1_bl.reshape(1, D)
    b32 = conv3_bl.reshape(1, D)

    cntp = _sc_counts(dst_flat)

    p1 = _sc_scatter_rows(x, srcp, dstp, zrows)[:, :N]
    h1, gm1, cvec = _conv1_call(p1, x, batch3, conv_Wl, bl2, conv_Wr, cntp)

    p2 = _sc_scatter_rows(h1, srcp, dstp, zrows)[:, :N]
    h2, gm2 = _conv2_call(p2, h1, batch3, conv1_Wl, b12, conv1_Wr, cvec)

    p3 = _sc_scatter_rows(h2, srcp, dstp, zrows)[:, :N]
    h3, gm3, q3 = _conv3_call(p3, h2, batch3, conv3_Wl, b32, conv3_Wr, cvec,
                              gcn_W)

    sp = _sc_edge_scalar(q3.reshape(N), src_flat, dst_flat)
    gm4 = _scorepool_call(h3, sp, cvec, q3, batch3, gcn_b.reshape(1, 1))

    logits, loss = _head_call(gm1, gm2, gm3, gm4, fc1_W,
                              fc1_b.reshape(1, -1), bn_g.reshape(1, -1),
                              bn_b.reshape(1, -1), fc2_W, fc2_b.reshape(1, -1),
                              y.reshape(G, 1))
    return logits, loss[0, 0]


# speedup vs baseline: 16.7889x; 16.7889x over previous
import functools

import jax
import jax.numpy as jnp
from jax import lax
from jax.experimental import pallas as pl
from jax.experimental.pallas import tpu as pltpu
from jax.experimental.pallas import tpu_sc as plsc

N = 10000
E = 320000
D = 128
G = 16
NC = 2
NS = 16
NW = NC * NS
NB = 10
BLK = 1024
NT = NB * BLK
CW = 128
NCHUNK = 80
EPW = NCHUNK * CW
RPT = NT // NS
NEG = -jnp.inf


@functools.cache
def _sc_kernels():
    mesh = plsc.VectorSubcoreMesh(core_axis_name="c", subcore_axis_name="s",
                                  num_cores=NC, num_subcores=NS)

    @functools.partial(
        pl.kernel,
        out_type=jax.ShapeDtypeStruct((NC, NT, D), jnp.float32),
        mesh=mesh,
        compiler_params=pltpu.CompilerParams(needs_layout_passes=False),
        scratch_types=[
            pltpu.VMEM((NCHUNK, CW), jnp.int32),
            pltpu.VMEM((NCHUNK, CW), jnp.int32),
            pltpu.VMEM((CW, D), jnp.float32),
            pltpu.VMEM_SHARED((NT, D), jnp.float32),
            pltpu.SemaphoreType.DMA,
        ],
    )
    def sc_scatter_rows(h_hbm, src_hbm, dst_hbm, z_hbm, out_hbm,
                        src_v, dst_v, rows_v, acc_sh, sem):
        c = lax.axis_index("c")
        s = lax.axis_index("s")
        w = s * NC + c
        pltpu.sync_copy(z_hbm, acc_sh.at[pl.ds(s * RPT, RPT)])
        pltpu.sync_copy(src_hbm.at[w], src_v)
        pltpu.sync_copy(dst_hbm.at[w], dst_v)
        plsc.subcore_barrier()

        def body(j, carry):
            pltpu.async_copy(h_hbm.at[src_v.at[j]], rows_v, sem).wait()
            pltpu.sync_copy(rows_v, acc_sh.at[dst_v.at[j]], add=True)
            return carry

        lax.fori_loop(0, NCHUNK, body, 0)
        plsc.subcore_barrier()
        pltpu.sync_copy(acc_sh.at[pl.ds(s * RPT, RPT)],
                        out_hbm.at[c, pl.ds(s * RPT, RPT)])

    @functools.partial(
        pl.kernel,
        out_type=jax.ShapeDtypeStruct((NW, NT), jnp.float32),
        mesh=mesh,
        compiler_params=pltpu.CompilerParams(needs_layout_passes=False),
        scratch_types=[
            pltpu.VMEM((EPW,), jnp.int32),
            pltpu.VMEM((NT,), jnp.float32),
        ],
    )
    def sc_counts(dst_hbm, out_hbm, dst_v, acc_v):
        c = lax.axis_index("c")
        s = lax.axis_index("s")
        w = s * NC + c
        pltpu.sync_copy(dst_hbm.at[w], dst_v)
        zeros16 = jnp.zeros((16,), jnp.float32)

        def zbody(i, carry):
            acc_v[pl.ds(i * 16, 16)] = zeros16
            return carry

        lax.fori_loop(0, NT // 16, zbody, 0)
        ones16 = jnp.ones((16,), jnp.float32)

        def body(j, carry):
            idx = dst_v[pl.ds(j * 16, 16)]
            plsc.addupdate_scatter(acc_v, [idx], ones16)
            return carry

        lax.fori_loop(0, EPW // 16, body, 0)
        pltpu.sync_copy(acc_v, out_hbm.at[w])

    @functools.partial(
        pl.kernel,
        out_type=jax.ShapeDtypeStruct((NW, NT), jnp.float32),
        mesh=mesh,
        compiler_params=pltpu.CompilerParams(needs_layout_passes=False),
        scratch_types=[
            pltpu.VMEM((NT,), jnp.float32),
            pltpu.VMEM((EPW,), jnp.int32),
            pltpu.VMEM((EPW,), jnp.int32),
            pltpu.VMEM((NT,), jnp.float32),
        ],
    )
    def sc_edge_scalar(q_hbm, src_hbm, dst_hbm, out_hbm,
                       q_v, src_v, dst_v, acc_v):
        c = lax.axis_index("c")
        s = lax.axis_index("s")
        w = s * NC + c
        pltpu.sync_copy(q_hbm, q_v)
        pltpu.sync_copy(src_hbm.at[w], src_v)
        pltpu.sync_copy(dst_hbm.at[w], dst_v)
        zeros16 = jnp.zeros((16,), jnp.float32)

        def zbody(i, carry):
            acc_v[pl.ds(i * 16, 16)] = zeros16
            return carry

        lax.fori_loop(0, NT // 16, zbody, 0)

        def body(j, carry):
            si = src_v[pl.ds(j * 16, 16)]
            di = dst_v[pl.ds(j * 16, 16)]
            vals = plsc.load_gather(q_v, [si])
            plsc.addupdate_scatter(acc_v, [di], vals)
            return carry

        lax.fori_loop(0, EPW // 16, body, 0)
        pltpu.sync_copy(acc_v, out_hbm.at[w])

    return sc_scatter_rows, sc_counts, sc_edge_scalar


def _gmax_update(gm_ref, hn, bb, r):
    prev = jnp.where(r == 0, jnp.full((G, D), NEG, jnp.float32), gm_ref[...])
    bcol = bb[:, None]
    cand = jnp.stack(
        [jnp.max(jnp.where(bcol == g, hn, NEG), axis=0)
         for g in range(G)], axis=0)
    gm_ref[...] = jnp.maximum(prev, cand)


def _make_conv_call(has_cnt, has_q):
    def body(*refs):
        it = iter(refs)
        p_ref = next(it); h_ref = next(it); b_ref = next(it)
        wl_ref = next(it); bl_ref = next(it); wr_ref = next(it)
        cnt_ref = next(it) if has_cnt else None
        c_ref = None if has_cnt else next(it)
        gcn_ref = next(it) if has_q else None
        hn_ref = next(it); gm_ref = next(it)
        c_out = next(it) if has_cnt else None
        q_out = next(it) if has_q else None

        r = pl.program_id(0)
        ssum = p_ref[0] + p_ref[1]
        if has_cnt:
            cvec = jnp.sum(cnt_ref[...], axis=0)
            c_out[0, 0] = cvec
        else:
            cvec = c_ref[0, 0]
        mean = ssum / jnp.maximum(cvec, 1.0)[:, None]
        hn = (jnp.dot(mean, wl_ref[...], preferred_element_type=jnp.float32)
              + bl_ref[...]
              + jnp.dot(h_ref[...], wr_ref[...],
                        preferred_element_type=jnp.float32))
        hn_ref[...] = hn
        _gmax_update(gm_ref, hn, b_ref[0, 0], r)
        if has_q:
            hl = jnp.dot(hn, gcn_ref[...], preferred_element_type=jnp.float32)
            dinv = lax.rsqrt(cvec + 1.0)
            q_out[0, 0] = hl[:, 0] * dinv

    in_specs = [
        pl.BlockSpec((2, BLK, D), lambda r: (0, r, 0)),
        pl.BlockSpec((BLK, D), lambda r: (r, 0)),
        pl.BlockSpec((1, 1, BLK), lambda r: (r, 0, 0)),
        pl.BlockSpec((D, D), lambda r: (0, 0)),
        pl.BlockSpec((1, D), lambda r: (0, 0)),
        pl.BlockSpec((D, D), lambda r: (0, 0)),
    ]
    if has_cnt:
        in_specs.append(pl.BlockSpec((NW, BLK), lambda r: (0, r)))
    else:
        in_specs.append(pl.BlockSpec((1, 1, BLK), lambda r: (r, 0, 0)))
    if has_q:
        in_specs.append(pl.BlockSpec((D, 1), lambda r: (0, 0)))

    out_shapes = [jax.ShapeDtypeStruct((NT, D), jnp.float32),
                  jax.ShapeDtypeStruct((G, D), jnp.float32)]
    out_specs = [pl.BlockSpec((BLK, D), lambda r: (r, 0)),
                 pl.BlockSpec((G, D), lambda r: (0, 0))]
    if has_cnt:
        out_shapes.append(jax.ShapeDtypeStruct((NB, 1, BLK), jnp.float32))
        out_specs.append(pl.BlockSpec((1, 1, BLK), lambda r: (r, 0, 0)))
    if has_q:
        out_shapes.append(jax.ShapeDtypeStruct((NB, 1, BLK), jnp.float32))
        out_specs.append(pl.BlockSpec((1, 1, BLK), lambda r: (r, 0, 0)))

    return pl.pallas_call(
        body,
        grid=(NB,),
        in_specs=in_specs,
        out_specs=tuple(out_specs),
        out_shape=tuple(out_shapes),
    )


_conv1_call = _make_conv_call(True, False)
_conv2_call = _make_conv_call(False, False)
_conv3_call = _make_conv_call(False, True)


def _score_body(sp_ref, c_ref, q_ref, b_ref, gb_ref, score_ref, keep_ref):
    es = jnp.sum(sp_ref[...], axis=0, keepdims=True)
    cc = c_ref[...]
    qq = q_ref[...]
    bb = b_ref[...]
    dinv = lax.rsqrt(cc + 1.0)
    sraw = dinv * (es + qq) + gb_ref[0, 0]
    masks = [bb == g for g in range(G)]
    smax_n = jnp.zeros_like(sraw)
    for g in range(G):
        mg = jnp.max(jnp.where(masks[g], sraw, NEG))
        smax_n = smax_n + jnp.where(masks[g], mg, 0.0)
    e = jnp.exp(sraw - smax_n)
    ssum_n = jnp.zeros_like(sraw)
    for g in range(G):
        sg = jnp.sum(jnp.where(masks[g], e, 0.0))
        ssum_n = ssum_n + jnp.where(masks[g], sg, 0.0)
    score = e / ssum_n
    scmax_n = jnp.zeros_like(sraw)
    for g in range(G):
        mg = jnp.max(jnp.where(masks[g], score, NEG))
        scmax_n = scmax_n + jnp.where(masks[g], mg, 0.0)
    scmin = jnp.minimum(scmax_n - 1e-7, 0.001)
    score_ref[...] = score
    keep_ref[...] = jnp.where(score > scmin, 1.0, 0.0)


_score_call = pl.pallas_call(
    _score_body,
    out_shape=(jax.ShapeDtypeStruct((1, NT), jnp.float32),
               jax.ShapeDtypeStruct((1, NT), jnp.float32)),
)


def _pool4_body(h3_ref, sc_ref, kp_ref, bc_ref, gm_ref):
    r = pl.program_id(0)
    masked = jnp.where(kp_ref[0] > 0.0, h3_ref[...] * sc_ref[0], NEG)
    prev = jnp.where(r == 0, jnp.full((G, D), NEG, jnp.float32), gm_ref[...])
    bcol = bc_ref[0]
    cand = jnp.stack(
        [jnp.max(jnp.where(bcol == g, masked, NEG), axis=0)
         for g in range(G)], axis=0)
    gm_ref[...] = jnp.maximum(prev, cand)


_pool4_call = pl.pallas_call(
    _pool4_body,
    grid=(NB,),
    in_specs=[
        pl.BlockSpec((BLK, D), lambda r: (r, 0)),
        pl.BlockSpec((1, BLK, 1), lambda r: (r, 0, 0)),
        pl.BlockSpec((1, BLK, 1), lambda r: (r, 0, 0)),
        pl.BlockSpec((1, BLK, 1), lambda r: (r, 0, 0)),
    ],
    out_specs=pl.BlockSpec((G, D), lambda r: (0, 0)),
    out_shape=jax.ShapeDtypeStruct((G, D), jnp.float32),
)


def _head_body(g1_ref, g2_ref, g3_ref, g4_ref, w1_ref, b1_ref, bg_ref, bb_ref,
               w2_ref, b2_ref, y_ref, logits_ref, loss_ref):
    z = jnp.concatenate(
        [g1_ref[...], g2_ref[...], g3_ref[...], g4_ref[...]], axis=1)
    z = jnp.dot(z, w1_ref[...], preferred_element_type=jnp.float32) + b1_ref[...]
    mu = jnp.mean(z, axis=0, keepdims=True)
    var = jnp.mean((z - mu) ** 2, axis=0, keepdims=True)
    z = (z - mu) / jnp.sqrt(var + 1e-5) * bg_ref[...] + bb_ref[...]
    z = jnp.maximum(z, 0.0)
    logits = jnp.dot(z, w2_ref[...], preferred_element_type=jnp.float32) + b2_ref[...]
    m = jnp.max(logits, axis=1, keepdims=True)
    lse = m + jnp.log(jnp.sum(jnp.exp(logits - m), axis=1, keepdims=True))
    logp = logits - lse
    oh = (y_ref[...] == lax.broadcasted_iota(jnp.int32, (1, 2), 1)
          ).astype(jnp.float32)
    loss = -jnp.mean(jnp.sum(logp * oh, axis=1))
    logits_ref[...] = logits
    loss_ref[...] = jnp.reshape(loss, (1, 1))


_head_call = pl.pallas_call(
    _head_body,
    out_shape=(jax.ShapeDtypeStruct((G, 2), jnp.float32),
               jax.ShapeDtypeStruct((1, 1), jnp.float32)),
)


def _pad_rows(a):
    return jnp.pad(a.reshape(NB, N // NB, D),
                   ((0, 0), (0, BLK - N // NB), (0, 0))).reshape(NT, D)


def kernel(x, edge_index, batch, y, conv_Wl, conv_bl, conv_Wr,
           conv1_Wl, conv1_bl, conv1_Wr, conv3_Wl, conv3_bl, conv3_Wr,
           gcn_W, gcn_b, fc1_W, fc1_b, bn_g, bn_b, fc2_W, fc2_b):
    src = edge_index[0]
    dst = edge_index[1]
    srcm = src + 24 * (src // 1000)
    dstm = dst + 24 * (dst // 1000)
    npad = NW * EPW - E
    k = jnp.arange(npad, dtype=jnp.int32)
    pslot = 1000 + (k % 24) + BLK * ((k // 24) % NB)
    sfull = jnp.concatenate([srcm, pslot])
    dfull = jnp.concatenate([dstm, pslot])
    src3 = sfull.reshape(NW, NCHUNK, CW)
    dst3 = dfull.reshape(NW, NCHUNK, CW)
    srcf = sfull.reshape(NW, EPW)
    dstf = dfull.reshape(NW, EPW)
    xp = _pad_rows(x)
    batch3 = jnp.pad(batch.reshape(NB, 1, N // NB),
                     ((0, 0), (0, 0), (0, BLK - N // NB)),
                     constant_values=G)
    zrows = jnp.zeros((RPT, D), jnp.float32)
    bl2 = conv_bl.reshape(1, D)
    b12 = conv1_bl.reshape(1, D)
    b32 = conv3_bl.reshape(1, D)

    _sc_scatter_rows, _sc_counts, _sc_edge_scalar = _sc_kernels()
    cntp = _sc_counts(dstf)

    p1 = _sc_scatter_rows(xp, src3, dst3, zrows)
    h1, gm1, cvec = _conv1_call(p1, xp, batch3, conv_Wl, bl2, conv_Wr, cntp)

    p2 = _sc_scatter_rows(h1, src3, dst3, zrows)
    h2, gm2 = _conv2_call(p2, h1, batch3, conv1_Wl, b12, conv1_Wr, cvec)

    p3 = _sc_scatter_rows(h2, src3, dst3, zrows)
    h3, gm3, q3 = _conv3_call(p3, h2, batch3, conv3_Wl, b32, conv3_Wr, cvec,
                              gcn_W)

    sp = _sc_edge_scalar(q3.reshape(NT), srcf, dstf)
    score, keep = _score_call(sp, cvec.reshape(1, NT), q3.reshape(1, NT),
                              batch3.reshape(1, NT), gcn_b.reshape(1, 1))
    gm4 = _pool4_call(h3, score.reshape(NB, BLK, 1), keep.reshape(NB, BLK, 1),
                      batch3.reshape(NB, BLK, 1))

    logits, loss = _head_call(gm1, gm2, gm3, gm4, fc1_W,
                              fc1_b.reshape(1, -1), bn_g.reshape(1, -1),
                              bn_b.reshape(1, -1), fc2_W, fc2_b.reshape(1, -1),
                              y.reshape(G, 1))
    return logits, loss[0, 0]


# speedup vs baseline: 20.9913x; 1.2503x over previous
import functools

import jax
import jax.numpy as jnp
from jax import lax
from jax.experimental import pallas as pl
from jax.experimental.pallas import tpu as pltpu
from jax.experimental.pallas import tpu_sc as plsc

N = 10000
E = 320000
D = 128
G = 16
NC = 2
NS = 16
NW = NC * NS
NB = 10
BLK = 1024
NT = NB * BLK
CW = 128
NCHUNK = 80
EPW = NCHUNK * CW
RPT = NT // NS
NBUF = 4
NEG = -jnp.inf


@functools.cache
def _sc_kernels():
    mesh = plsc.VectorSubcoreMesh(core_axis_name="c", subcore_axis_name="s",
                                  num_cores=NC, num_subcores=NS)

    @functools.partial(
        pl.kernel,
        out_type=jax.ShapeDtypeStruct((NC, NT, D), jnp.float32),
        mesh=mesh,
        compiler_params=pltpu.CompilerParams(needs_layout_passes=False),
        scratch_types=[
            pltpu.VMEM((NCHUNK, CW), jnp.int32),
            pltpu.VMEM((2, CW), jnp.int32),
            pltpu.VMEM((2, CW, D), jnp.float32),
            pltpu.VMEM_SHARED((NT, D), jnp.float32),
            pltpu.SemaphoreType.DMA,
            pltpu.SemaphoreType.DMA,
            pltpu.SemaphoreType.DMA,
            pltpu.SemaphoreType.DMA,
        ],
    )
    def sc_scatter_rows(h_hbm, src_hbm, dst_hbm, z_hbm, out_hbm,
                        src_v, didx, rows_v, acc_sh, g0, g1, d0, d1):
        gsem = (g0, g1)
        dsem = (d0, d1)
        c = lax.axis_index("c")
        s = lax.axis_index("s")
        w = s * NC + c
        pltpu.sync_copy(z_hbm, acc_sh.at[pl.ds(s * RPT, RPT)])
        pltpu.sync_copy(src_hbm.at[w], src_v)
        plsc.subcore_barrier()

        def gather(j, b):
            pltpu.async_copy(h_hbm.at[src_v.at[j]], rows_v.at[b], gsem[b])

        def wait_gather(j, b):
            pltpu.make_async_copy(h_hbm.at[src_v.at[j]], rows_v.at[b],
                                  gsem[b]).wait()

        def fetch_didx(j, b):
            pltpu.async_copy(dst_hbm.at[w, j], didx.at[b], dsem[b])

        def wait_didx(j, b):
            pltpu.make_async_copy(dst_hbm.at[w, j], didx.at[b],
                                  dsem[b]).wait()

        def scatter(j, b):
            wait_didx(j, b)
            pltpu.sync_copy(rows_v.at[b], acc_sh.at[didx.at[b]], add=True)

        fetch_didx(0, 0)
        fetch_didx(1, 1)
        gather(0, 0)

        def body(it, carry):
            for b in range(2):
                j = it * 2 + b
                wait_gather(j, b)
                gather(j + 1, 1 - b)
                scatter(j, b)
                fetch_didx(j + 2, b)
            return carry

        lax.fori_loop(0, NCHUNK // 2 - 1, body, 0)
        wait_gather(NCHUNK - 2, 0)
        gather(NCHUNK - 1, 1)
        scatter(NCHUNK - 2, 0)
        wait_gather(NCHUNK - 1, 1)
        scatter(NCHUNK - 1, 1)
        plsc.subcore_barrier()
        pltpu.sync_copy(acc_sh.at[pl.ds(s * RPT, RPT)],
                        out_hbm.at[c, pl.ds(s * RPT, RPT)])

    @functools.partial(
        pl.kernel,
        out_type=jax.ShapeDtypeStruct((NW, NT), jnp.float32),
        mesh=mesh,
        compiler_params=pltpu.CompilerParams(needs_layout_passes=False),
        scratch_types=[
            pltpu.VMEM((EPW,), jnp.int32),
            pltpu.VMEM((NT,), jnp.float32),
        ],
    )
    def sc_counts(dst_hbm, out_hbm, dst_v, acc_v):
        c = lax.axis_index("c")
        s = lax.axis_index("s")
        w = s * NC + c
        pltpu.sync_copy(dst_hbm.at[w], dst_v)
        zeros16 = jnp.zeros((16,), jnp.float32)

        def zbody(i, carry):
            acc_v[pl.ds(i * 16, 16)] = zeros16
            return carry

        lax.fori_loop(0, NT // 16, zbody, 0)
        ones16 = jnp.ones((16,), jnp.float32)

        def body(j, carry):
            idx = dst_v[pl.ds(j * 16, 16)]
            plsc.addupdate_scatter(acc_v, [idx], ones16)
            return carry

        lax.fori_loop(0, EPW // 16, body, 0)
        pltpu.sync_copy(acc_v, out_hbm.at[w])

    @functools.partial(
        pl.kernel,
        out_type=jax.ShapeDtypeStruct((NW, NT), jnp.float32),
        mesh=mesh,
        compiler_params=pltpu.CompilerParams(needs_layout_passes=False),
        scratch_types=[
            pltpu.VMEM((NT,), jnp.float32),
            pltpu.VMEM((EPW,), jnp.int32),
            pltpu.VMEM((EPW,), jnp.int32),
            pltpu.VMEM((NT,), jnp.float32),
        ],
    )
    def sc_edge_scalar(q_hbm, src_hbm, dst_hbm, out_hbm,
                       q_v, src_v, dst_v, acc_v):
        c = lax.axis_index("c")
        s = lax.axis_index("s")
        w = s * NC + c
        pltpu.sync_copy(q_hbm, q_v)
        pltpu.sync_copy(src_hbm.at[w], src_v)
        pltpu.sync_copy(dst_hbm.at[w], dst_v)
        zeros16 = jnp.zeros((16,), jnp.float32)

        def zbody(i, carry):
            acc_v[pl.ds(i * 16, 16)] = zeros16
            return carry

        lax.fori_loop(0, NT // 16, zbody, 0)

        def body(j, carry):
            si = src_v[pl.ds(j * 16, 16)]
            di = dst_v[pl.ds(j * 16, 16)]
            vals = plsc.load_gather(q_v, [si])
            plsc.addupdate_scatter(acc_v, [di], vals)
            return carry

        lax.fori_loop(0, EPW // 16, body, 0)
        pltpu.sync_copy(acc_v, out_hbm.at[w])

    return sc_scatter_rows, sc_counts, sc_edge_scalar


def _gmax_update(gm_ref, hn, bb, r):
    prev = jnp.where(r == 0, jnp.full((G, D), NEG, jnp.float32), gm_ref[...])
    bcol = bb[:, None]
    cand = jnp.stack(
        [jnp.max(jnp.where(bcol == g, hn, NEG), axis=0)
         for g in range(G)], axis=0)
    gm_ref[...] = jnp.maximum(prev, cand)


def _make_conv_call(has_cnt, has_q):
    def body(*refs):
        it = iter(refs)
        p_ref = next(it); h_ref = next(it); b_ref = next(it)
        wl_ref = next(it); bl_ref = next(it); wr_ref = next(it)
        cnt_ref = next(it) if has_cnt else None
        c_ref = None if has_cnt else next(it)
        gcn_ref = next(it) if has_q else None
        hn_ref = next(it); gm_ref = next(it)
        c_out = next(it) if has_cnt else None
        q_out = next(it) if has_q else None

        r = pl.program_id(0)
        ssum = p_ref[0] + p_ref[1]
        if has_cnt:
            cvec = jnp.sum(cnt_ref[...], axis=0)
            c_out[0, 0] = cvec
        else:
            cvec = c_ref[0, 0]
        mean = ssum / jnp.maximum(cvec, 1.0)[:, None]
        hn = (jnp.dot(mean, wl_ref[...], preferred_element_type=jnp.float32)
              + bl_ref[...]
              + jnp.dot(h_ref[...], wr_ref[...],
                        preferred_element_type=jnp.float32))
        hn_ref[...] = hn
        _gmax_update(gm_ref, hn, b_ref[0, 0], r)
        if has_q:
            hl = jnp.dot(hn, gcn_ref[...], preferred_element_type=jnp.float32)
            dinv = lax.rsqrt(cvec + 1.0)
            q_out[0, 0] = hl[:, 0] * dinv

    in_specs = [
        pl.BlockSpec((2, BLK, D), lambda r: (0, r, 0)),
        pl.BlockSpec((BLK, D), lambda r: (r, 0)),
        pl.BlockSpec((1, 1, BLK), lambda r: (r, 0, 0)),
        pl.BlockSpec((D, D), lambda r: (0, 0)),
        pl.BlockSpec((1, D), lambda r: (0, 0)),
        pl.BlockSpec((D, D), lambda r: (0, 0)),
    ]
    if has_cnt:
        in_specs.append(pl.BlockSpec((NW, BLK), lambda r: (0, r)))
    else:
        in_specs.append(pl.BlockSpec((1, 1, BLK), lambda r: (r, 0, 0)))
    if has_q:
        in_specs.append(pl.BlockSpec((D, 1), lambda r: (0, 0)))

    out_shapes = [jax.ShapeDtypeStruct((NT, D), jnp.float32),
                  jax.ShapeDtypeStruct((G, D), jnp.float32)]
    out_specs = [pl.BlockSpec((BLK, D), lambda r: (r, 0)),
                 pl.BlockSpec((G, D), lambda r: (0, 0))]
    if has_cnt:
        out_shapes.append(jax.ShapeDtypeStruct((NB, 1, BLK), jnp.float32))
        out_specs.append(pl.BlockSpec((1, 1, BLK), lambda r: (r, 0, 0)))
    if has_q:
        out_shapes.append(jax.ShapeDtypeStruct((NB, 1, BLK), jnp.float32))
        out_specs.append(pl.BlockSpec((1, 1, BLK), lambda r: (r, 0, 0)))

    return pl.pallas_call(
        body,
        grid=(NB,),
        in_specs=in_specs,
        out_specs=tuple(out_specs),
        out_shape=tuple(out_shapes),
    )


_conv1_call = _make_conv_call(True, False)
_conv2_call = _make_conv_call(False, False)
_conv3_call = _make_conv_call(False, True)


def _score_body(sp_ref, c_ref, q_ref, b_ref, gb_ref, score_ref, keep_ref):
    es = jnp.sum(sp_ref[...], axis=0, keepdims=True)
    cc = c_ref[...]
    qq = q_ref[...]
    bb = b_ref[...]
    dinv = lax.rsqrt(cc + 1.0)
    sraw = dinv * (es + qq) + gb_ref[0, 0]
    masks = [bb == g for g in range(G)]
    smax_n = jnp.zeros_like(sraw)
    for g in range(G):
        mg = jnp.max(jnp.where(masks[g], sraw, NEG))
        smax_n = smax_n + jnp.where(masks[g], mg, 0.0)
    e = jnp.exp(sraw - smax_n)
    ssum_n = jnp.zeros_like(sraw)
    for g in range(G):
        sg = jnp.sum(jnp.where(masks[g], e, 0.0))
        ssum_n = ssum_n + jnp.where(masks[g], sg, 0.0)
    score = e / ssum_n
    scmax_n = jnp.zeros_like(sraw)
    for g in range(G):
        mg = jnp.max(jnp.where(masks[g], score, NEG))
        scmax_n = scmax_n + jnp.where(masks[g], mg, 0.0)
    scmin = jnp.minimum(scmax_n - 1e-7, 0.001)
    score_ref[...] = score
    keep_ref[...] = jnp.where(score > scmin, 1.0, 0.0)


_score_call = pl.pallas_call(
    _score_body,
    out_shape=(jax.ShapeDtypeStruct((1, NT), jnp.float32),
               jax.ShapeDtypeStruct((1, NT), jnp.float32)),
)


def _pool4_body(h3_ref, sc_ref, kp_ref, bc_ref, gm_ref):
    r = pl.program_id(0)
    masked = jnp.where(kp_ref[0] > 0.0, h3_ref[...] * sc_ref[0], NEG)
    prev = jnp.where(r == 0, jnp.full((G, D), NEG, jnp.float32), gm_ref[...])
    bcol = bc_ref[0]
    cand = jnp.stack(
        [jnp.max(jnp.where(bcol == g, masked, NEG), axis=0)
         for g in range(G)], axis=0)
    gm_ref[...] = jnp.maximum(prev, cand)


_pool4_call = pl.pallas_call(
    _pool4_body,
    grid=(NB,),
    in_specs=[
        pl.BlockSpec((BLK, D), lambda r: (r, 0)),
        pl.BlockSpec((1, BLK, 1), lambda r: (r, 0, 0)),
        pl.BlockSpec((1, BLK, 1), lambda r: (r, 0, 0)),
        pl.BlockSpec((1, BLK, 1), lambda r: (r, 0, 0)),
    ],
    out_specs=pl.BlockSpec((G, D), lambda r: (0, 0)),
    out_shape=jax.ShapeDtypeStruct((G, D), jnp.float32),
)


def _head_body(g1_ref, g2_ref, g3_ref, g4_ref, w1_ref, b1_ref, bg_ref, bb_ref,
               w2_ref, b2_ref, y_ref, logits_ref, loss_ref):
    z = jnp.concatenate(
        [g1_ref[...], g2_ref[...], g3_ref[...], g4_ref[...]], axis=1)
    z = jnp.dot(z, w1_ref[...], preferred_element_type=jnp.float32) + b1_ref[...]
    mu = jnp.mean(z, axis=0, keepdims=True)
    var = jnp.mean((z - mu) ** 2, axis=0, keepdims=True)
    z = (z - mu) / jnp.sqrt(var + 1e-5) * bg_ref[...] + bb_ref[...]
    z = jnp.maximum(z, 0.0)
    logits = jnp.dot(z, w2_ref[...], preferred_element_type=jnp.float32) + b2_ref[...]
    m = jnp.max(logits, axis=1, keepdims=True)
    lse = m + jnp.log(jnp.sum(jnp.exp(logits - m), axis=1, keepdims=True))
    logp = logits - lse
    oh = (y_ref[...] == lax.broadcasted_iota(jnp.int32, (1, 2), 1)
          ).astype(jnp.float32)
    loss = -jnp.mean(jnp.sum(logp * oh, axis=1))
    logits_ref[...] = logits
    loss_ref[...] = jnp.reshape(loss, (1, 1))


_head_call = pl.pallas_call(
    _head_body,
    out_shape=(jax.ShapeDtypeStruct((G, 2), jnp.float32),
               jax.ShapeDtypeStruct((1, 1), jnp.float32)),
)


def _pad_rows(a):
    return jnp.pad(a.reshape(NB, N // NB, D),
                   ((0, 0), (0, BLK - N // NB), (0, 0))).reshape(NT, D)


def kernel(x, edge_index, batch, y, conv_Wl, conv_bl, conv_Wr,
           conv1_Wl, conv1_bl, conv1_Wr, conv3_Wl, conv3_bl, conv3_Wr,
           gcn_W, gcn_b, fc1_W, fc1_b, bn_g, bn_b, fc2_W, fc2_b):
    src = edge_index[0]
    dst = edge_index[1]
    srcm = src + 24 * (src // 1000)
    dstm = dst + 24 * (dst // 1000)
    npad = NW * EPW - E
    k = jnp.arange(npad, dtype=jnp.int32)
    pslot = 1000 + (k % 24) + BLK * ((k // 24) % NB)
    sfull = jnp.concatenate([srcm, pslot])
    dfull = jnp.concatenate([dstm, pslot])
    src3 = sfull.reshape(NW, NCHUNK, CW)
    dst3 = dfull.reshape(NW, NCHUNK, CW)
    srcf = sfull.reshape(NW, EPW)
    dstf = dfull.reshape(NW, EPW)
    xp = _pad_rows(x)
    batch3 = jnp.pad(batch.reshape(NB, 1, N // NB),
                     ((0, 0), (0, 0), (0, BLK - N // NB)),
                     constant_values=G)
    zrows = jnp.zeros((RPT, D), jnp.float32)
    bl2 = conv_bl.reshape(1, D)
    b12 = conv1_bl.reshape(1, D)
    b32 = conv3_bl.reshape(1, D)

    _sc_scatter_rows, _sc_counts, _sc_edge_scalar = _sc_kernels()
    cntp = _sc_counts(dstf)

    p1 = _sc_scatter_rows(xp, src3, dst3, zrows)
    h1, gm1, cvec = _conv1_call(p1, xp, batch3, conv_Wl, bl2, conv_Wr, cntp)

    p2 = _sc_scatter_rows(h1, src3, dst3, zrows)
    h2, gm2 = _conv2_call(p2, h1, batch3, conv1_Wl, b12, conv1_Wr, cvec)

    p3 = _sc_scatter_rows(h2, src3, dst3, zrows)
    h3, gm3, q3 = _conv3_call(p3, h2, batch3, conv3_Wl, b32, conv3_Wr, cvec,
                              gcn_W)

    sp = _sc_edge_scalar(q3.reshape(NT), srcf, dstf)
    score, keep = _score_call(sp, cvec.reshape(1, NT), q3.reshape(1, NT),
                              batch3.reshape(1, NT), gcn_b.reshape(1, 1))
    gm4 = _pool4_call(h3, score.reshape(NB, BLK, 1), keep.reshape(NB, BLK, 1),
                      batch3.reshape(NB, BLK, 1))

    logits, loss = _head_call(gm1, gm2, gm3, gm4, fc1_W,
                              fc1_b.reshape(1, -1), bn_g.reshape(1, -1),
                              bn_b.reshape(1, -1), fc2_W, fc2_b.reshape(1, -1),
                              y.reshape(G, 1))
    return logits, loss[0, 0]
